# baseline (speedup 1.0000x reference)
import jax
import jax.numpy as jnp
from jax.experimental import pallas as pl
from jax.experimental.pallas import tpu as pltpu


def _conv1_body(x_ref, w1_ref, b1_ref, o_ref):
    x = x_ref[0]
    taps = [x[kh:kh + 126, kw:kw + 171]
            for kh in range(3) for kw in range(3)]
    for co in range(24):
        y = taps[0] * w1_ref[co, 0]
        for t in range(1, 9):
            y = y + taps[t] * w1_ref[co, t]
        y = jnp.max(y.reshape(63, 2, 171), axis=1)
        yt = y.T
        yt = jnp.max(yt[:168].reshape(42, 4, 63), axis=1)
        o_ref[0, co] = jnp.maximum(yt + b1_ref[co], 0.0)


def _conv_block(xin, w_ref, b_ref, ho, wo, cin, cout, pool):
    acc = jnp.zeros((ho * wo, cout), jnp.float32)
    for kh in range(3):
        for kw in range(3):
            p = xin[kh:kh + ho, kw:kw + wo, :].reshape(ho * wo, cin)
            acc = acc + jnp.dot(p, w_ref[kh * 3 + kw],
                                preferred_element_type=jnp.float32)
    y = jnp.maximum(acc + b_ref[...], 0.0).reshape(ho, wo, cout)
    if pool == 1:
        return y
    hp, wp = ho // 2, wo // 2
    y = y[:hp * 2, :wp * 2]
    y = jnp.max(y.reshape(hp, 2, wp * 2, cout), axis=1)
    y = jnp.max(y.reshape(hp, wp, 2, cout), axis=2)
    return y


def _convs_body(x_ref, w2_ref, b2_ref, w3_ref, b3_ref, w4_ref, b4_ref, o_ref):
    x1 = x_ref[0]
    x2 = _conv_block(x1, w2_ref, b2_ref, 61, 40, 24, 48, 2)
    x3 = _conv_block(x2, w3_ref, b3_ref, 28, 18, 48, 64, 2)
    x4 = _conv_block(x3, w4_ref, b4_ref, 12, 7, 64, 64, 1)
    o_ref[0] = x4


def _fc_softmax_body(x_ref, w_ref, b_ref, o_ref):
    logits = jnp.dot(x_ref[...], w_ref[...],
                     preferred_element_type=jnp.float32) + b_ref[...]
    m = jnp.max(logits, axis=0, keepdims=True)
    e = jnp.exp(logits - m)
    o_ref[...] = e / jnp.sum(e, axis=0, keepdims=True)


@jax.jit
def kernel(w1, s1, b1, w2, s2, b2, w3, s3, b3, w4, s4, b4, wfc, bfc, x):
    n = x.shape[0]
    x0 = x[:, 0].astype(jnp.float32)
    w1s = w1 * s1[:, None]
    w2s = w2 * s2[None, :, :]
    w3s = w3 * s3[None, :, :]
    w4s = w4 * s4[None, :, :]
    a1 = pl.pallas_call(
        _conv1_body,
        out_shape=jax.ShapeDtypeStruct((n, 24, 42, 63), jnp.float32),
        grid=(n,),
        in_specs=[
            pl.BlockSpec((1, 128, 173), lambda i: (i, 0, 0)),
            pl.BlockSpec(memory_space=pltpu.MemorySpace.SMEM),
            pl.BlockSpec(memory_space=pltpu.MemorySpace.SMEM),
        ],
        out_specs=pl.BlockSpec((1, 24, 42, 63), lambda i: (i, 0, 0, 0)),
        compiler_params=pltpu.CompilerParams(
            dimension_semantics=("parallel",)),
    )(x0, w1s, b1)
    x1 = jnp.transpose(a1, (0, 3, 2, 1))
    x4 = pl.pallas_call(
        _convs_body,
        out_shape=jax.ShapeDtypeStruct((n, 12, 7, 64), jnp.float32),
        grid=(n,),
        in_specs=[
            pl.BlockSpec((1, 63, 42, 24), lambda i: (i, 0, 0, 0)),
            pl.BlockSpec((9, 24, 48), lambda i: (0, 0, 0)),
            pl.BlockSpec((1, 48), lambda i: (0, 0)),
            pl.BlockSpec((9, 48, 64), lambda i: (0, 0, 0)),
            pl.BlockSpec((1, 64), lambda i: (0, 0)),
            pl.BlockSpec((9, 64, 64), lambda i: (0, 0, 0)),
            pl.BlockSpec((1, 64), lambda i: (0, 0)),
        ],
        out_specs=pl.BlockSpec((1, 12, 7, 64), lambda i: (i, 0, 0, 0)),
        compiler_params=pltpu.CompilerParams(
            dimension_semantics=("parallel",),
            vmem_limit_bytes=48 * 1024 * 1024),
    )(x1, w2s, b2, w3s, b3, w4s, b4)
    xf = x4.reshape(n, 12 * 7 * 64)
    return pl.pallas_call(
        _fc_softmax_body,
        out_shape=jax.ShapeDtypeStruct((n, 10), jnp.float32),
        grid=(1,),
        in_specs=[
            pl.BlockSpec((n, 5376), lambda i: (0, 0)),
            pl.BlockSpec((5376, 10), lambda i: (0, 0)),
            pl.BlockSpec((1, 10), lambda i: (0, 0)),
        ],
        out_specs=pl.BlockSpec((n, 10), lambda i: (0, 0)),
    )(xf, wfc, bfc)


# speedup vs baseline: 1.0148x; 1.0148x over previous
import jax
import jax.numpy as jnp
from jax.experimental import pallas as pl
from jax.experimental.pallas import tpu as pltpu


def _conv1_body(x_ref, w1_ref, b1_ref, o_ref):
    for b in range(2):
        x = x_ref[b]
        taps = [x[kh:kh + 126, kw:kw + 171]
                for kh in range(3) for kw in range(3)]
        for co in range(24):
            y = taps[0] * w1_ref[co, 0]
            for t in range(1, 9):
                y = y + taps[t] * w1_ref[co, t]
            y = jnp.maximum(y + b1_ref[co], 0.0)
            y = jnp.max(y.reshape(63, 2, 171), axis=1)
            yt = y.T
            yt = jnp.max(yt[:168].reshape(42, 4, 63), axis=1)
            o_ref[b, co] = yt


def _conv_block(xin, w_ref, b_ref, ho, wo, cin, cout, pool):
    acc = jnp.zeros((ho * wo, cout), jnp.float32)
    for kh in range(3):
        for kw in range(3):
            p = xin[kh:kh + ho, kw:kw + wo, :].reshape(ho * wo, cin)
            acc = acc + jnp.dot(p, w_ref[kh * 3 + kw],
                                preferred_element_type=jnp.float32)
    y = jnp.maximum(acc + b_ref[...], 0.0).reshape(ho, wo, cout)
    if pool == 1:
        return y
    hp, wp = ho // 2, wo // 2
    y = y[:hp * 2, :wp * 2]
    y = jnp.max(y.reshape(hp, 2, wp * 2, cout), axis=1)
    y = jnp.max(y.reshape(hp, wp, 2, cout), axis=2)
    return y


def _convs_body(x_ref, w2_ref, b2_ref, w3_ref, b3_ref, w4_ref, b4_ref, o_ref):
    x1 = x_ref[0]
    x2 = _conv_block(x1, w2_ref, b2_ref, 61, 40, 24, 48, 2)
    x3 = _conv_block(x2, w3_ref, b3_ref, 28, 18, 48, 64, 2)
    x4 = _conv_block(x3, w4_ref, b4_ref, 12, 7, 64, 64, 1)
    o_ref[0] = x4


def _fc_softmax_body(x_ref, w_ref, b_ref, o_ref):
    logits = jnp.dot(x_ref[...], w_ref[...],
                     preferred_element_type=jnp.float32) + b_ref[...]
    m = jnp.max(logits, axis=0, keepdims=True)
    e = jnp.exp(logits - m)
    o_ref[...] = e / jnp.sum(e, axis=0, keepdims=True)


@jax.jit
def kernel(w1, s1, b1, w2, s2, b2, w3, s3, b3, w4, s4, b4, wfc, bfc, x):
    n = x.shape[0]
    x0 = x[:, 0].astype(jnp.float32)
    w1s = w1 * s1[:, None]
    w2s = w2 * s2[None, :, :]
    w3s = w3 * s3[None, :, :]
    w4s = w4 * s4[None, :, :]
    a1 = pl.pallas_call(
        _conv1_body,
        out_shape=jax.ShapeDtypeStruct((n, 24, 42, 63), jnp.float32),
        grid=(n // 2,),
        in_specs=[
            pl.BlockSpec((2, 128, 173), lambda i: (i, 0, 0)),
            pl.BlockSpec(memory_space=pltpu.MemorySpace.SMEM),
            pl.BlockSpec(memory_space=pltpu.MemorySpace.SMEM),
        ],
        out_specs=pl.BlockSpec((2, 24, 42, 63), lambda i: (i, 0, 0, 0)),
        compiler_params=pltpu.CompilerParams(
            dimension_semantics=("parallel",)),
    )(x0, w1s, b1)
    x1 = jnp.transpose(a1, (0, 3, 2, 1))
    x4 = pl.pallas_call(
        _convs_body,
        out_shape=jax.ShapeDtypeStruct((n, 12, 7, 64), jnp.float32),
        grid=(n,),
        in_specs=[
            pl.BlockSpec((1, 63, 42, 24), lambda i: (i, 0, 0, 0)),
            pl.BlockSpec((9, 24, 48), lambda i: (0, 0, 0)),
            pl.BlockSpec((1, 48), lambda i: (0, 0)),
            pl.BlockSpec((9, 48, 64), lambda i: (0, 0, 0)),
            pl.BlockSpec((1, 64), lambda i: (0, 0)),
            pl.BlockSpec((9, 64, 64), lambda i: (0, 0, 0)),
            pl.BlockSpec((1, 64), lambda i: (0, 0)),
        ],
        out_specs=pl.BlockSpec((1, 12, 7, 64), lambda i: (i, 0, 0, 0)),
        compiler_params=pltpu.CompilerParams(
            dimension_semantics=("parallel",),
            vmem_limit_bytes=48 * 1024 * 1024),
    )(x1, w2s, b2, w3s, b3, w4s, b4)
    xf = x4.reshape(n, 12 * 7 * 64)
    return pl.pallas_call(
        _fc_softmax_body,
        out_shape=jax.ShapeDtypeStruct((n, 10), jnp.float32),
        grid=(1,),
        in_specs=[
            pl.BlockSpec((n, 5376), lambda i: (0, 0)),
            pl.BlockSpec((5376, 10), lambda i: (0, 0)),
            pl.BlockSpec((1, 10), lambda i: (0, 0)),
        ],
        out_specs=pl.BlockSpec((n, 10), lambda i: (0, 0)),
    )(xf, wfc, bfc)


# speedup vs baseline: 1.0212x; 1.0063x over previous
import jax
import jax.numpy as jnp
from jax.experimental import pallas as pl
from jax.experimental.pallas import tpu as pltpu


def _conv1_body(x_ref, w1_ref, b1_ref, o_ref):
    for b in range(2):
        x = x_ref[b]
        taps = [x[kh:kh + 126, kw:kw + 171]
                for kh in range(3) for kw in range(3)]
        for co in range(24):
            y = taps[0] * w1_ref[co, 0]
            for t in range(1, 9):
                y = y + taps[t] * w1_ref[co, t]
            y = jnp.maximum(y + b1_ref[co], 0.0)
            y = jnp.max(y.reshape(63, 2, 171), axis=1)
            yt = y.T
            yt = jnp.max(yt[:168].reshape(42, 4, 63), axis=1)
            o_ref[b, co] = yt


def _conv_block(xin, w_ref, b_ref, ho, wo, cin, cout, pool):
    acc = jnp.zeros((ho * wo, cout), jnp.float32)
    for kh in range(3):
        for kw in range(3):
            p = xin[kh:kh + ho, kw:kw + wo, :].reshape(ho * wo, cin)
            acc = acc + jnp.dot(p, w_ref[kh * 3 + kw],
                                preferred_element_type=jnp.float32)
    y = jnp.maximum(acc + b_ref[...], 0.0).reshape(ho, wo, cout)
    if pool == 1:
        return y
    hp, wp = ho // 2, wo // 2
    y = y[:hp * 2, :wp * 2]
    y = jnp.max(y.reshape(hp, 2, wp * 2, cout), axis=1)
    y = jnp.max(y.reshape(hp, wp, 2, cout), axis=2)
    return y


def _convs_body(x_ref, w2_ref, b2_ref, w3_ref, b3_ref, w4_ref, b4_ref, o_ref):
    for b in range(2):
        x1 = x_ref[b]
        x2 = _conv_block(x1, w2_ref, b2_ref, 61, 40, 24, 48, 2)
        x3 = _conv_block(x2, w3_ref, b3_ref, 28, 18, 48, 64, 2)
        x4 = _conv_block(x3, w4_ref, b4_ref, 12, 7, 64, 64, 1)
        o_ref[b] = x4


def _fc_softmax_body(x_ref, w_ref, b_ref, o_ref):
    logits = jnp.dot(x_ref[...], w_ref[...],
                     preferred_element_type=jnp.float32) + b_ref[...]
    m = jnp.max(logits, axis=0, keepdims=True)
    e = jnp.exp(logits - m)
    o_ref[...] = e / jnp.sum(e, axis=0, keepdims=True)


@jax.jit
def kernel(w1, s1, b1, w2, s2, b2, w3, s3, b3, w4, s4, b4, wfc, bfc, x):
    n = x.shape[0]
    x0 = x[:, 0].astype(jnp.float32)
    w1s = w1 * s1[:, None]
    w2s = w2 * s2[None, :, :]
    w3s = w3 * s3[None, :, :]
    w4s = w4 * s4[None, :, :]
    a1 = pl.pallas_call(
        _conv1_body,
        out_shape=jax.ShapeDtypeStruct((n, 24, 42, 63), jnp.float32),
        grid=(n // 2,),
        in_specs=[
            pl.BlockSpec((2, 128, 173), lambda i: (i, 0, 0)),
            pl.BlockSpec(memory_space=pltpu.MemorySpace.SMEM),
            pl.BlockSpec(memory_space=pltpu.MemorySpace.SMEM),
        ],
        out_specs=pl.BlockSpec((2, 24, 42, 63), lambda i: (i, 0, 0, 0)),
        compiler_params=pltpu.CompilerParams(
            dimension_semantics=("parallel",)),
    )(x0, w1s, b1)
    x1 = jnp.transpose(a1, (0, 3, 2, 1))
    x4 = pl.pallas_call(
        _convs_body,
        out_shape=jax.ShapeDtypeStruct((n, 12, 7, 64), jnp.float32),
        grid=(n // 2,),
        in_specs=[
            pl.BlockSpec((2, 63, 42, 24), lambda i: (i, 0, 0, 0)),
            pl.BlockSpec((9, 24, 48), lambda i: (0, 0, 0)),
            pl.BlockSpec((1, 48), lambda i: (0, 0)),
            pl.BlockSpec((9, 48, 64), lambda i: (0, 0, 0)),
            pl.BlockSpec((1, 64), lambda i: (0, 0)),
            pl.BlockSpec((9, 64, 64), lambda i: (0, 0, 0)),
            pl.BlockSpec((1, 64), lambda i: (0, 0)),
        ],
        out_specs=pl.BlockSpec((2, 12, 7, 64), lambda i: (i, 0, 0, 0)),
        compiler_params=pltpu.CompilerParams(
            dimension_semantics=("parallel",),
            vmem_limit_bytes=48 * 1024 * 1024),
    )(x1, w2s, b2, w3s, b3, w4s, b4)
    xf = x4.reshape(n, 12 * 7 * 64)
    return pl.pallas_call(
        _fc_softmax_body,
        out_shape=jax.ShapeDtypeStruct((n, 10), jnp.float32),
        grid=(1,),
        in_specs=[
            pl.BlockSpec((n, 5376), lambda i: (0, 0)),
            pl.BlockSpec((5376, 10), lambda i: (0, 0)),
            pl.BlockSpec((1, 10), lambda i: (0, 0)),
        ],
        out_specs=pl.BlockSpec((n, 10), lambda i: (0, 0)),
    )(xf, wfc, bfc)
